# baseline (speedup 1.0000x reference)
import jax
import jax.numpy as jnp
from jax import lax
from jax.experimental import pallas as pl
from jax.experimental.pallas import tpu as pltpu
from jax.experimental.pallas import tpu_sc as plsc

_K = 9
_KD = 2 * _K
_ROWS = 2 * 8 * 10000
_NW = 32
_RPW = _ROWS // _NW
_NP = 1008
_NCH = -(-_RPW // _NP)
_RSTARTS = [min(i * _NP, _RPW - _NP) for i in range(_NCH)]
_OUT_CH = _NP * _K


def _deinterleave_body(in_hbm4, out_hbm4, in_a, in_b, out_a, out_b,
                       in_sem, out_sem):
    c = lax.axis_index("c")
    s = lax.axis_index("s")
    wid = s * 2 + c
    a = wid // 16
    b = (wid // 2) % 8
    p_base = (wid % 2) * _RPW
    iota = lax.iota(jnp.int32, 16)
    in_bufs = (in_a, in_b)
    out_bufs = (out_a, out_b)

    def copy_in(i):
        p0 = p_base + _RSTARTS[i]
        return pltpu.async_copy(
            in_hbm4.at[a, b, pl.ds(p0, _NP), :], in_bufs[i % 2],
            in_sem.at[i % 2])

    def copy_out(i):
        p0 = p_base + _RSTARTS[i]
        return pltpu.async_copy(
            out_bufs[i % 2], out_hbm4.at[a, b, pl.ds(p0, _NP), :],
            out_sem.at[i % 2])

    in_copies = [copy_in(0)]
    out_copies = [None] * _NCH
    for i in range(_NCH):
        if i + 1 < _NCH:
            in_copies.append(copy_in(i + 1))
        if i >= 2:
            out_copies[i - 2].wait()
        in_copies[i].wait()

        buf_in = in_bufs[i % 2]
        buf_out = out_bufs[i % 2]

        @plsc.parallel_loop(0, _NP, 16, unroll=7)
        def _gather(j):
            rows = iota + j
            for k in range(_K):
                v = plsc.load_gather(
                    buf_in, [rows, jnp.full((16,), 2 * k, jnp.int32)])
                plsc.store_scatter(
                    buf_out, [rows, jnp.full((16,), k, jnp.int32)], v)

        out_copies[i] = copy_out(i)

    out_copies[_NCH - 2].wait()
    out_copies[_NCH - 1].wait()


def kernel(edge_index):
    sh = edge_index.shape
    out = pl.kernel(
        _deinterleave_body,
        out_type=jax.ShapeDtypeStruct((sh[0], sh[1], sh[2], _K), jnp.int32),
        mesh=plsc.VectorSubcoreMesh(core_axis_name="c", subcore_axis_name="s"),
        scratch_types=[
            pltpu.VMEM((_NP, _KD), jnp.int32),
            pltpu.VMEM((_NP, _KD), jnp.int32),
            pltpu.VMEM((_NP, _K), jnp.int32),
            pltpu.VMEM((_NP, _K), jnp.int32),
            pltpu.SemaphoreType.DMA((2,)),
            pltpu.SemaphoreType.DMA((2,)),
        ],
        compiler_params=pltpu.CompilerParams(needs_layout_passes=False, use_tc_tiling_on_sc=False),
    )(edge_index)
    return out


# speedup vs baseline: 7.2184x; 7.2184x over previous
import jax
import jax.numpy as jnp
from jax import lax
from jax.experimental import pallas as pl
from jax.experimental.pallas import tpu as pltpu
from jax.experimental.pallas import tpu_sc as plsc

_K = 9
_B = 8
_NPTS = 10000
_HALF = _NPTS // 2
_NW = 32
_PIECES = 2 * _K * _B * 2
_T = _PIECES // _NW
_NBUF = 4


def _slab_copy_body(in_hbm, out_hbm, b0, b1, b2, b3, isems, osems):
    c = lax.axis_index("c")
    s = lax.axis_index("s")
    wid = s * 2 + c
    bufs = (b0, b1, b2, b3)

    def coords(t):
        pid = wid + t * _NW
        row = pid // 2
        half = pid % 2
        a = row // (_K * _B)
        k = (row // _B) % _K
        b = row % _B
        return a, k, b, half * _HALF

    def copy_in(t):
        a, k, b, off = coords(t)
        return pltpu.async_copy(
            in_hbm.at[a, 2 * k, b, pl.ds(off, _HALF)], bufs[t % _NBUF],
            isems.at[t % _NBUF])

    def copy_out(t):
        a, k, b, off = coords(t)
        return pltpu.async_copy(
            bufs[t % _NBUF], out_hbm.at[a, k, b, pl.ds(off, _HALF)],
            osems.at[t % _NBUF])

    hin = [None] * _T
    hout = [None] * _T
    hin[0] = copy_in(0)
    for t in range(_T):
        r = t + 1
        if r < _T:
            if r >= _NBUF:
                hout[r - _NBUF].wait()
            hin[r] = copy_in(r)
        hin[t].wait()
        hout[t] = copy_out(t)
    for t in range(max(0, _T - _NBUF), _T):
        hout[t].wait()


def kernel(edge_index):
    x = jnp.transpose(edge_index, (0, 3, 1, 2))
    out_t = pl.kernel(
        _slab_copy_body,
        out_type=jax.ShapeDtypeStruct((2, _K, _B, _NPTS), jnp.int32),
        mesh=plsc.VectorSubcoreMesh(core_axis_name="c", subcore_axis_name="s"),
        scratch_types=[
            pltpu.VMEM((_HALF,), jnp.int32),
            pltpu.VMEM((_HALF,), jnp.int32),
            pltpu.VMEM((_HALF,), jnp.int32),
            pltpu.VMEM((_HALF,), jnp.int32),
            pltpu.SemaphoreType.DMA((_NBUF,)),
            pltpu.SemaphoreType.DMA((_NBUF,)),
        ],
        compiler_params=pltpu.CompilerParams(needs_layout_passes=False,
                                             use_tc_tiling_on_sc=False),
    )(x)
    return jnp.transpose(out_t, (0, 2, 3, 1))


# speedup vs baseline: 7.2184x; 1.0000x over previous
import jax
import jax.numpy as jnp
from jax import lax
from jax.experimental import pallas as pl
from jax.experimental.pallas import tpu as pltpu
from jax.experimental.pallas import tpu_sc as plsc

_K = 9
_B = 8
_NPTS = 10000
_HALF = _NPTS // 2
_NW = 32
_PIECES = 2 * _K * _B * 2
_T = _PIECES // _NW
_NBUF = 4


def _slab_copy_body(in_hbm, out_hbm, b0, b1, b2, b3, isems, osems):
    c = lax.axis_index("c")
    s = lax.axis_index("s")
    wid = s * 2 + c
    bufs = (b0, b1, b2, b3)

    def coords(t):
        pid = wid + t * _NW
        row = pid // 2
        half = pid % 2
        a = row // (_K * _B)
        k = (row // _B) % _K
        b = row % _B
        return a, k, b, half * _HALF

    def copy_in(t):
        a, k, b, off = coords(t)
        return pltpu.async_copy(
            in_hbm.at[a, 2 * k, b, pl.ds(off, _HALF)], bufs[t % _NBUF],
            isems.at[t % _NBUF])

    def copy_out(t):
        a, k, b, off = coords(t)
        return pltpu.async_copy(
            bufs[t % _NBUF], out_hbm.at[a, k, b, pl.ds(off, _HALF)],
            osems.at[t % _NBUF])

    hin = [None] * _T
    hout = [None] * _T
    hin[0] = copy_in(0)
    for t in range(_T):
        r = t + 1
        if r < _T:
            if r >= _NBUF:
                hout[r - _NBUF].wait()
            hin[r] = copy_in(r)
        hin[t].wait()
        hout[t] = copy_out(t)
    for t in range(max(0, _T - _NBUF), _T):
        hout[t].wait()


def kernel(edge_index):
    x = jnp.transpose(edge_index, (0, 3, 1, 2))
    out_t = pl.kernel(
        _slab_copy_body,
        out_type=jax.ShapeDtypeStruct((2, _K, _B, _NPTS), jnp.int32),
        mesh=plsc.VectorSubcoreMesh(core_axis_name="c", subcore_axis_name="s"),
        scratch_types=[
            pltpu.VMEM((_HALF,), jnp.int32),
            pltpu.VMEM((_HALF,), jnp.int32),
            pltpu.VMEM((_HALF,), jnp.int32),
            pltpu.VMEM((_HALF,), jnp.int32),
            pltpu.SemaphoreType.DMA((_NBUF,)),
            pltpu.SemaphoreType.DMA((_NBUF,)),
        ],
        compiler_params=pltpu.CompilerParams(needs_layout_passes=False,
                                             use_tc_tiling_on_sc=False,
                                             skip_device_barrier=True),
    )(x)
    return jnp.transpose(out_t, (0, 2, 3, 1))
